# baseline (speedup 1.0000x reference)
import functools

import jax
import jax.numpy as jnp
from jax.experimental import pallas as pl
from jax.experimental.pallas import tpu as pltpu


def _pick_bm(n, cap=400):
    for bm in (cap, 200, 80, 40, 16, 8):
        if bm <= cap and n % bm == 0 and bm % 8 == 0:
            return bm
    return n


def _fused_kernel(
    nb, bm,
    x_ref, w0_ref, wcat_ref, adj_ref, eps_ref,
    z_ref, m_ref, s_ref,
    xw0_ref, h_ref, hw_ref,
):
    latent = m_ref.shape[1]
    i = pl.program_id(0)

    @pl.when(i == 0)
    def _():
        xw0_ref[...] = jnp.dot(
            x_ref[...], w0_ref[...], preferred_element_type=jnp.float32
        ).astype(jnp.bfloat16)

    @pl.when(i < nb)
    def _():
        h_ref[pl.ds(i * bm, bm), :] = jnp.tanh(
            jnp.dot(
                adj_ref[...].astype(jnp.bfloat16),
                xw0_ref[...],
                preferred_element_type=jnp.float32,
            )
        ).astype(jnp.bfloat16)

    @pl.when(i == nb)
    def _():
        hw_ref[...] = jnp.dot(
            h_ref[...], wcat_ref[...], preferred_element_type=jnp.float32
        ).astype(jnp.bfloat16)

    @pl.when(i >= nb)
    def _():
        acc = jnp.dot(
            adj_ref[...].astype(jnp.bfloat16),
            hw_ref[...],
            preferred_element_type=jnp.float32,
        )
        m = acc[:, :latent]
        s = jnp.maximum(acc[:, latent:], 0.0) + 0.0001
        m_ref[...] = m
        s_ref[...] = s
        z_ref[...] = eps_ref[...] * s + m


def kernel(adj, x, W0, Wm, Ws):
    n, d_in = x.shape
    hidden = W0.shape[1]
    latent = Wm.shape[1]
    bm = _pick_bm(n)
    nb = n // bm

    wcat = jnp.concatenate([Wm, Ws], axis=1)
    eps = jax.random.normal(jax.random.key(42), (n, latent), dtype=jnp.float32)

    out_sds = jax.ShapeDtypeStruct((n, latent), jnp.float32)
    ph2_spec = pl.BlockSpec(
        (bm, latent), lambda i: (jax.lax.rem(i, nb), 0)
    )
    z, m_q_z, std_q_z = pl.pallas_call(
        functools.partial(_fused_kernel, nb, bm),
        grid=(2 * nb,),
        in_specs=[
            pl.BlockSpec((n, d_in), lambda i: (0, 0)),
            pl.BlockSpec((d_in, hidden), lambda i: (0, 0)),
            pl.BlockSpec((hidden, 2 * latent), lambda i: (0, 0)),
            pl.BlockSpec((bm, n), lambda i: (jax.lax.rem(i, nb), 0)),
            ph2_spec,
        ],
        out_specs=[ph2_spec, ph2_spec, ph2_spec],
        out_shape=[out_sds, out_sds, out_sds],
        scratch_shapes=[
            pltpu.VMEM((n, hidden), jnp.bfloat16),
            pltpu.VMEM((n, hidden), jnp.bfloat16),
            pltpu.VMEM((n, 2 * latent), jnp.bfloat16),
        ],
        compiler_params=pltpu.CompilerParams(
            dimension_semantics=("arbitrary",),
        ),
    )(x, W0, wcat, adj, eps)

    return (z, m_q_z, std_q_z)


# speedup vs baseline: 1.1175x; 1.1175x over previous
import functools

import jax
import jax.numpy as jnp
from jax.experimental import pallas as pl
from jax.experimental.pallas import tpu as pltpu


def _pick_bm(n, cap=200):
    for bm in (cap, 200, 80, 40, 16, 8):
        if bm <= cap and n % bm == 0 and bm % 8 == 0:
            return bm
    return n


def _fused_kernel(
    nb, bm,
    x_ref, w0_ref, wcat_ref, adj_ref, eps_ref,
    z_ref, m_ref, s_ref,
    xw0_ref, h_ref, hw_ref,
):
    latent = hw_ref.shape[1] // 2
    i = pl.program_id(0)

    @pl.when(i == 0)
    def _():
        xw0_ref[...] = jnp.dot(
            x_ref[...], w0_ref[...], preferred_element_type=jnp.float32
        ).astype(jnp.bfloat16)

    @pl.when(i < nb)
    def _():
        h_ref[pl.ds(i * bm, bm), :] = jnp.tanh(
            jnp.dot(
                adj_ref[...].astype(jnp.bfloat16),
                xw0_ref[...],
                preferred_element_type=jnp.float32,
            )
        ).astype(jnp.bfloat16)

    @pl.when(i == nb)
    def _():
        hw_ref[...] = jnp.dot(
            h_ref[...], wcat_ref[...], preferred_element_type=jnp.float32
        ).astype(jnp.bfloat16)

    @pl.when(i >= nb)
    def _():
        row0 = (i - nb) * bm
        acc = jnp.dot(
            adj_ref[...].astype(jnp.bfloat16),
            hw_ref[...],
            preferred_element_type=jnp.float32,
        )
        m = acc[:, :latent]
        s = jnp.maximum(acc[:, latent:], 0.0) + 0.0001
        m_ref[pl.ds(row0, bm), :] = m
        s_ref[pl.ds(row0, bm), :] = s
        z_ref[pl.ds(row0, bm), :] = eps_ref[pl.ds(row0, bm), :] * s + m


def kernel(adj, x, W0, Wm, Ws):
    n, d_in = x.shape
    hidden = W0.shape[1]
    latent = Wm.shape[1]
    bm = _pick_bm(n)
    nb = n // bm

    wcat = jnp.concatenate([Wm, Ws], axis=1)
    with jax.ensure_compile_time_eval():
        eps = jax.random.normal(
            jax.random.key(42), (n, latent), dtype=jnp.float32
        )

    out_sds = jax.ShapeDtypeStruct((n, latent), jnp.float32)
    resident = pl.BlockSpec((n, latent), lambda i: (0, 0))
    z, m_q_z, std_q_z = pl.pallas_call(
        functools.partial(_fused_kernel, nb, bm),
        grid=(2 * nb,),
        in_specs=[
            pl.BlockSpec((n, d_in), lambda i: (0, 0)),
            pl.BlockSpec((d_in, hidden), lambda i: (0, 0)),
            pl.BlockSpec((hidden, 2 * latent), lambda i: (0, 0)),
            pl.BlockSpec((bm, n), lambda i: (jax.lax.rem(i, nb), 0)),
            resident,
        ],
        out_specs=[resident, resident, resident],
        out_shape=[out_sds, out_sds, out_sds],
        scratch_shapes=[
            pltpu.VMEM((n, hidden), jnp.bfloat16),
            pltpu.VMEM((n, hidden), jnp.bfloat16),
            pltpu.VMEM((n, 2 * latent), jnp.bfloat16),
        ],
        compiler_params=pltpu.CompilerParams(
            dimension_semantics=("arbitrary",),
        ),
    )(x, W0, wcat, adj, eps)

    return (z, m_q_z, std_q_z)
